# baseline (speedup 1.0000x reference)
import jax
import jax.numpy as jnp
from jax import lax
from jax.experimental import pallas as pl
from jax.experimental.pallas import tpu as pltpu
from jax.experimental.pallas import tpu_sc as plsc

B = 4096
D = 384
C = 100000
MOM = 0.9

NTILES = 16
N_EL = B // NTILES
CH = 128
NSUB = N_EL // CH
LANES = 16


def _sc_scatter_phase(l_hbm, rep_o, l2d, vals2d):
    tid = lax.axis_index("s")
    base = tid * N_EL

    for c in range(NSUB):
        pltpu.sync_copy(l_hbm.at[pl.ds(base + c * CH, CH)], l2d.at[c])
    for c in range(NSUB):
        for k in range(CH // LANES):
            vals2d[c, pl.ds(k * LANES, LANES)] = (
                lax.iota(jnp.int32, LANES) + (base + c * CH + k * LANES))

    for c in range(NSUB):
        pltpu.sync_copy(vals2d.at[c], rep_o.at[l2d.at[c]])


def _sc_gather_phase(l_hbm, rep_hbm, ci_hbm, cs_hbm,
                     r_o, cir_o, csr_o,
                     l2d, r2d, gbuf0, gbuf1, sem0, sem1):
    tid = lax.axis_index("s")
    base = tid * N_EL

    for c in range(NSUB):
        pltpu.sync_copy(l_hbm.at[pl.ds(base + c * CH, CH)], l2d.at[c])

    for c in range(NSUB):
        pltpu.sync_copy(rep_hbm.at[l2d.at[c]], r2d.at[c])
        pltpu.sync_copy(r2d.at[c], r_o.at[pl.ds(base + c * CH, CH)])

    cp0 = pltpu.async_copy(ci_hbm.at[l2d.at[0]], gbuf0, sem0)
    cp1 = pltpu.async_copy(cs_hbm.at[l2d.at[0]], gbuf1, sem1)
    cp0.wait()
    pltpu.sync_copy(gbuf0, cir_o.at[pl.ds(base, CH)])
    cp0 = pltpu.async_copy(ci_hbm.at[l2d.at[1]], gbuf0, sem0)
    cp1.wait()
    pltpu.sync_copy(gbuf1, csr_o.at[pl.ds(base, CH)])
    cp1 = pltpu.async_copy(cs_hbm.at[l2d.at[1]], gbuf1, sem1)
    cp0.wait()
    pltpu.sync_copy(gbuf0, cir_o.at[pl.ds(base + CH, CH)])
    cp1.wait()
    pltpu.sync_copy(gbuf1, csr_o.at[pl.ds(base + CH, CH)])


def _sc_call(l, ci, cs):
    mesh = plsc.VectorSubcoreMesh(core_axis_name="c", subcore_axis_name="s",
                                  num_cores=1)
    scatter_fn = pl.kernel(
        _sc_scatter_phase,
        out_type=jax.ShapeDtypeStruct((C,), jnp.int32),
        mesh=mesh,
        scratch_types=[
            pltpu.VMEM((NSUB, CH), jnp.int32),
            pltpu.VMEM((NSUB, CH), jnp.int32),
        ],
    )
    rep = scatter_fn(l)
    gather_fn = pl.kernel(
        _sc_gather_phase,
        out_type=(
            jax.ShapeDtypeStruct((B,), jnp.int32),
            jax.ShapeDtypeStruct((B, D), jnp.float32),
            jax.ShapeDtypeStruct((B, D), jnp.float32),
        ),
        mesh=mesh,
        scratch_types=[
            pltpu.VMEM((NSUB, CH), jnp.int32),
            pltpu.VMEM((NSUB, CH), jnp.int32),
            pltpu.VMEM((CH, D), jnp.float32),
            pltpu.VMEM((CH, D), jnp.float32),
            pltpu.SemaphoreType.DMA,
            pltpu.SemaphoreType.DMA,
        ],
    )
    r, cir, csr = gather_fn(l, rep, ci, cs)
    return r, cir, csr


RB = 512
NBLK = B // RB


def _tc_phase(rcol_ref, rrow_ref, x_ref, ci_ref, cs_ref, out_ref, sums):
    blk = pl.program_id(0)

    @pl.when(blk == 0)
    def _():
        sums[0] = 0.0
        sums[1] = 0.0

    rr = rcol_ref[...]
    mm = rr == rrow_ref[...]
    cnt = jnp.sum(mm.astype(jnp.float32), axis=1, keepdims=True)
    acc = jax.lax.dot_general(
        mm.astype(jnp.bfloat16), x_ref[...],
        (((1,), (0,)), ((), ())), preferred_element_type=jnp.float32)
    mean = acc * (1.0 / jnp.maximum(cnt, 1.0))
    upd = ci_ref[...] * MOM + mean * (1.0 - MOM)
    n2 = jnp.sum(upd * upd, axis=1, keepdims=True)
    uh = upd * jnp.where(n2 > 0, lax.rsqrt(n2), 1.0)
    diff = uh - cs_ref[...]
    d2 = jnp.sum(diff * diff, axis=1, keepdims=True)
    gidx = blk * RB + lax.broadcasted_iota(jnp.int32, (RB, 1), 0)
    m = rr == gidx
    sums[0] += jnp.sum(jnp.where(m, d2, 0.0))
    sums[1] += jnp.sum(jnp.where(m, 1.0, 0.0))

    @pl.when(blk == NBLK - 1)
    def _():
        out_ref[...] = (sums[0] / jnp.maximum(sums[1], 1.0)).reshape(1, 1)


def _tc_call(r, x_bf, cir, csr):
    return pl.pallas_call(
        _tc_phase,
        grid=(NBLK,),
        in_specs=[
            pl.BlockSpec((RB, 1), lambda i: (i, 0)),
            pl.BlockSpec((1, B), lambda i: (0, 0)),
            pl.BlockSpec((B, D), lambda i: (0, 0)),
            pl.BlockSpec((RB, D), lambda i: (i, 0)),
            pl.BlockSpec((RB, D), lambda i: (i, 0)),
        ],
        out_specs=pl.BlockSpec((1, 1), lambda i: (0, 0)),
        out_shape=jax.ShapeDtypeStruct((1, 1), jnp.float32),
        scratch_shapes=[pltpu.SMEM((2,), jnp.float32)],
    )(r.reshape(B, 1), r.reshape(1, B), x_bf, cir, csr)


def kernel(x, l, center_img, center_skt):
    r, cir, csr = _sc_call(l, center_img, center_skt)
    loss = _tc_call(r, x.astype(jnp.bfloat16), cir, csr)
    return loss.reshape(())


# speedup vs baseline: 1.1737x; 1.1737x over previous
import jax
import jax.numpy as jnp
from jax import lax
from jax.experimental import pallas as pl
from jax.experimental.pallas import tpu as pltpu
from jax.experimental.pallas import tpu_sc as plsc

B = 4096
D = 384
MOM = 0.9

NC = 2
NS = 16
NW = NC * NS
N_EL = B // NW
CH = 128
LANES = 16


def _sc_gather_phase(l_hbm, ci_hbm, cs_hbm, cir_o, csr_o,
                     l2d, gbuf0, gbuf1, sem0, sem1):
    wid = lax.axis_index("s") * NC + lax.axis_index("c")
    base = wid * N_EL

    pltpu.sync_copy(l_hbm.at[pl.ds(base, CH)], l2d.at[0])

    cp0 = pltpu.async_copy(ci_hbm.at[l2d.at[0]], gbuf0, sem0)
    cp1 = pltpu.async_copy(cs_hbm.at[l2d.at[0]], gbuf1, sem1)
    cp0.wait()
    pltpu.sync_copy(gbuf0, cir_o.at[pl.ds(base, CH)])
    cp1.wait()
    pltpu.sync_copy(gbuf1, csr_o.at[pl.ds(base, CH)])


def _sc_call(l, ci, cs):
    mesh = plsc.VectorSubcoreMesh(core_axis_name="c", subcore_axis_name="s",
                                  num_cores=NC)
    fn = pl.kernel(
        _sc_gather_phase,
        out_type=(
            jax.ShapeDtypeStruct((B, D), jnp.float32),
            jax.ShapeDtypeStruct((B, D), jnp.float32),
        ),
        mesh=mesh,
        scratch_types=[
            pltpu.VMEM((1, CH), jnp.int32),
            pltpu.VMEM((CH, D), jnp.float32),
            pltpu.VMEM((CH, D), jnp.float32),
            pltpu.SemaphoreType.DMA,
            pltpu.SemaphoreType.DMA,
        ],
    )
    return fn(l, ci, cs)


RB = 512
NBLK = B // RB


def _tc_phase(lcol_ref, lrow_ref, x_ref, ci_ref, cs_ref, out_ref, sums):
    blk = pl.program_id(0)

    @pl.when(blk == 0)
    def _():
        sums[0] = 0.0
        sums[1] = 0.0

    lc = lcol_ref[...]
    lr = lrow_ref[...]
    mm = lc == lr
    cnt = jnp.sum(mm.astype(jnp.float32), axis=1, keepdims=True)
    jrow = lax.broadcasted_iota(jnp.int32, (1, B), 1)
    minidx = jnp.min(jnp.where(mm, jrow, B), axis=1, keepdims=True)
    acc = jax.lax.dot_general(
        mm.astype(jnp.bfloat16), x_ref[...],
        (((1,), (0,)), ((), ())), preferred_element_type=jnp.float32)
    mean = acc * (1.0 / jnp.maximum(cnt, 1.0))
    upd = ci_ref[...] * MOM + mean * (1.0 - MOM)
    n2 = jnp.sum(upd * upd, axis=1, keepdims=True)
    uh = upd * jnp.where(n2 > 0, lax.rsqrt(n2), 1.0)
    diff = uh - cs_ref[...]
    d2 = jnp.sum(diff * diff, axis=1, keepdims=True)
    gidx = blk * RB + lax.broadcasted_iota(jnp.int32, (RB, 1), 0)
    first = minidx == gidx
    sums[0] += jnp.sum(jnp.where(first, d2, 0.0))
    sums[1] += jnp.sum(jnp.where(first, 1.0, 0.0))

    @pl.when(blk == NBLK - 1)
    def _():
        out_ref[...] = (sums[0] / jnp.maximum(sums[1], 1.0)).reshape(1, 1)


def _tc_call(l, x_bf, cir, csr):
    return pl.pallas_call(
        _tc_phase,
        grid=(NBLK,),
        in_specs=[
            pl.BlockSpec((RB, 1), lambda i: (i, 0)),
            pl.BlockSpec((1, B), lambda i: (0, 0)),
            pl.BlockSpec((B, D), lambda i: (0, 0)),
            pl.BlockSpec((RB, D), lambda i: (i, 0)),
            pl.BlockSpec((RB, D), lambda i: (i, 0)),
        ],
        out_specs=pl.BlockSpec((1, 1), lambda i: (0, 0)),
        out_shape=jax.ShapeDtypeStruct((1, 1), jnp.float32),
        scratch_shapes=[pltpu.SMEM((2,), jnp.float32)],
    )(l.reshape(B, 1), l.reshape(1, B), x_bf, cir, csr)


def kernel(x, l, center_img, center_skt):
    cir, csr = _sc_call(l, center_img, center_skt)
    loss = _tc_call(l, x.astype(jnp.bfloat16), cir, csr)
    return loss.reshape(())


# speedup vs baseline: 1.3715x; 1.1686x over previous
import jax
import jax.numpy as jnp
from jax import lax
from jax.experimental import pallas as pl
from jax.experimental.pallas import tpu as pltpu
from jax.experimental.pallas import tpu_sc as plsc

B = 4096
D = 384
MOM = 0.9

NC = 2
NS = 16
NW = NC * NS
N_EL = B // NW
CH = 128
LANES = 16


def _sc_gather_phase(l_hbm, ci_hbm, cs_hbm, cir_o, csr_o,
                     l2d, gbuf0, gbuf1, sem0, sem1):
    wid = lax.axis_index("s") * NC + lax.axis_index("c")
    base = wid * N_EL

    pltpu.sync_copy(l_hbm.at[pl.ds(base, CH)], l2d.at[0])

    cp0 = pltpu.async_copy(ci_hbm.at[l2d.at[0]], gbuf0, sem0)
    cp1 = pltpu.async_copy(cs_hbm.at[l2d.at[0]], gbuf1, sem1)
    cp0.wait()
    pltpu.sync_copy(gbuf0, cir_o.at[pl.ds(base, CH)])
    cp1.wait()
    pltpu.sync_copy(gbuf1, csr_o.at[pl.ds(base, CH)])


def _sc_call(l, ci, cs):
    mesh = plsc.VectorSubcoreMesh(core_axis_name="c", subcore_axis_name="s",
                                  num_cores=NC)
    fn = pl.kernel(
        _sc_gather_phase,
        out_type=(
            jax.ShapeDtypeStruct((B, D), jnp.float32),
            jax.ShapeDtypeStruct((B, D), jnp.float32),
        ),
        mesh=mesh,
        scratch_types=[
            pltpu.VMEM((1, CH), jnp.int32),
            pltpu.VMEM((CH, D), jnp.float32),
            pltpu.VMEM((CH, D), jnp.float32),
            pltpu.SemaphoreType.DMA,
            pltpu.SemaphoreType.DMA,
        ],
    )
    return fn(l, ci, cs)


RB = 512
NBLK = B // RB


def _tc_phase(lcol_ref, lrow_ref, xa_ref, ci_ref, cs_ref, out_ref, sums):
    blk = pl.program_id(0)

    @pl.when(blk == 0)
    def _():
        sums[0] = 0.0
        sums[1] = 0.0

    lc = lcol_ref[...]
    lr = lrow_ref[...]
    mm = lc == lr
    acc_aug = jax.lax.dot_general(
        mm.astype(jnp.bfloat16), xa_ref[...],
        (((1,), (0,)), ((), ())), preferred_element_type=jnp.float32)
    acc = acc_aug[:, :D]
    cnt = acc_aug[:, D:D + 1]
    inv = 1.0 / jnp.maximum(cnt, 1.0)
    mean = acc * inv
    upd = ci_ref[...] * MOM + mean * (1.0 - MOM)
    n2 = jnp.sum(upd * upd, axis=1, keepdims=True)
    uh = upd * jnp.where(n2 > 0, lax.rsqrt(n2), 1.0)
    diff = uh - cs_ref[...]
    d2 = jnp.sum(diff * diff, axis=1, keepdims=True)
    sums[0] += jnp.sum(d2 * inv)
    sums[1] += jnp.sum(inv)

    @pl.when(blk == NBLK - 1)
    def _():
        out_ref[...] = (sums[0] / jnp.maximum(sums[1], 1.0)).reshape(1, 1)


DA = D + 1


def _tc_call(l, xa, cir, csr):
    return pl.pallas_call(
        _tc_phase,
        grid=(NBLK,),
        in_specs=[
            pl.BlockSpec((RB, 1), lambda i: (i, 0)),
            pl.BlockSpec((1, B), lambda i: (0, 0)),
            pl.BlockSpec((B, DA), lambda i: (0, 0)),
            pl.BlockSpec((RB, D), lambda i: (i, 0)),
            pl.BlockSpec((RB, D), lambda i: (i, 0)),
        ],
        out_specs=pl.BlockSpec((1, 1), lambda i: (0, 0)),
        out_shape=jax.ShapeDtypeStruct((1, 1), jnp.float32),
        scratch_shapes=[pltpu.SMEM((2,), jnp.float32)],
    )(l.reshape(B, 1), l.reshape(1, B), xa, cir, csr)


def kernel(x, l, center_img, center_skt):
    cir, csr = _sc_call(l, center_img, center_skt)
    xa = jnp.concatenate(
        [x.astype(jnp.bfloat16), jnp.ones((B, 1), jnp.bfloat16)], axis=1)
    loss = _tc_call(l, xa, cir, csr)
    return loss.reshape(())


# speedup vs baseline: 1.4097x; 1.0279x over previous
import jax
import jax.numpy as jnp
from jax import lax
from jax.experimental import pallas as pl
from jax.experimental.pallas import tpu as pltpu
from jax.experimental.pallas import tpu_sc as plsc

B = 4096
D = 384
MOM = 0.9

NC = 2
NS = 16
NW = NC * NS
N_EL = B // NW
CH = 128
LANES = 16


def _sc_gather_phase(l_hbm, ci_hbm, cs_hbm, cir_o, csr_o,
                     l2d, gbuf0, gbuf1, sem0, sem1):
    wid = lax.axis_index("s") * NC + lax.axis_index("c")
    base = wid * N_EL

    pltpu.sync_copy(l_hbm.at[pl.ds(base, CH)], l2d.at[0])

    cp0 = pltpu.async_copy(ci_hbm.at[l2d.at[0]], gbuf0, sem0)
    cp1 = pltpu.async_copy(cs_hbm.at[l2d.at[0]], gbuf1, sem1)
    cp0.wait()
    pltpu.sync_copy(gbuf0, cir_o.at[pl.ds(base, CH)])
    cp1.wait()
    pltpu.sync_copy(gbuf1, csr_o.at[pl.ds(base, CH)])


def _sc_call(l, ci, cs):
    mesh = plsc.VectorSubcoreMesh(core_axis_name="c", subcore_axis_name="s",
                                  num_cores=NC)
    fn = pl.kernel(
        _sc_gather_phase,
        out_type=(
            jax.ShapeDtypeStruct((B, D), jnp.float32),
            jax.ShapeDtypeStruct((B, D), jnp.float32),
        ),
        mesh=mesh,
        scratch_types=[
            pltpu.VMEM((1, CH), jnp.int32),
            pltpu.VMEM((CH, D), jnp.float32),
            pltpu.VMEM((CH, D), jnp.float32),
            pltpu.SemaphoreType.DMA,
            pltpu.SemaphoreType.DMA,
        ],
    )
    return fn(l, ci, cs)


RB = 512
NBLK = B // RB


def _tc_phase(lcol_ref, lrow_ref, x_ref, ci_ref, cs_ref, out_ref,
              sums, xa_s):
    blk = pl.program_id(0)

    @pl.when(blk == 0)
    def _():
        sums[0] = 0.0
        sums[1] = 0.0
        xa_s[:, :D] = x_ref[...].astype(jnp.bfloat16)
        xa_s[:, D:DA] = jnp.ones((B, 1), jnp.bfloat16)

    lc = lcol_ref[...]
    lr = lrow_ref[...]
    mm = lc == lr
    acc_aug = jax.lax.dot_general(
        mm.astype(jnp.bfloat16), xa_s[...],
        (((1,), (0,)), ((), ())), preferred_element_type=jnp.float32)
    acc = acc_aug[:, :D]
    cnt = acc_aug[:, D:D + 1]
    inv = 1.0 / jnp.maximum(cnt, 1.0)
    mean = acc * inv
    upd = ci_ref[...] * MOM + mean * (1.0 - MOM)
    n2 = jnp.sum(upd * upd, axis=1, keepdims=True)
    uh = upd * jnp.where(n2 > 0, lax.rsqrt(n2), 1.0)
    diff = uh - cs_ref[...]
    d2 = jnp.sum(diff * diff, axis=1, keepdims=True)
    sums[0] += jnp.sum(d2 * inv)
    sums[1] += jnp.sum(inv)

    @pl.when(blk == NBLK - 1)
    def _():
        out_ref[...] = (sums[0] / jnp.maximum(sums[1], 1.0)).reshape(1, 1)


DA = D + 1


def _tc_call(l, x, cir, csr):
    return pl.pallas_call(
        _tc_phase,
        grid=(NBLK,),
        in_specs=[
            pl.BlockSpec((RB, 1), lambda i: (i, 0)),
            pl.BlockSpec((1, B), lambda i: (0, 0)),
            pl.BlockSpec((B, D), lambda i: (0, 0)),
            pl.BlockSpec((RB, D), lambda i: (i, 0)),
            pl.BlockSpec((RB, D), lambda i: (i, 0)),
        ],
        out_specs=pl.BlockSpec((1, 1), lambda i: (0, 0)),
        out_shape=jax.ShapeDtypeStruct((1, 1), jnp.float32),
        scratch_shapes=[pltpu.SMEM((2,), jnp.float32),
                        pltpu.VMEM((B, DA), jnp.bfloat16)],
    )(l.reshape(B, 1), l.reshape(1, B), x, cir, csr)


def kernel(x, l, center_img, center_skt):
    cir, csr = _sc_call(l, center_img, center_skt)
    loss = _tc_call(l, x, cir, csr)
    return loss.reshape(())


# speedup vs baseline: 1.5023x; 1.0657x over previous
import jax
import jax.numpy as jnp
from jax import lax
from jax.experimental import pallas as pl
from jax.experimental.pallas import tpu as pltpu
from jax.experimental.pallas import tpu_sc as plsc

B = 4096
D = 384
MOM = 0.9

NC = 2
NS = 16
NW = NC * NS
N_EL = B // NW
CH = 128
LANES = 16


def _sc_gather_phase(l_hbm, ci_hbm, cs_hbm, cir_o, csr_o,
                     l2d, gbuf0, gbuf1, sem0, sem1):
    wid = lax.axis_index("s") * NC + lax.axis_index("c")
    base = wid * N_EL

    pltpu.sync_copy(l_hbm.at[pl.ds(base, CH)], l2d.at[0])

    cp0 = pltpu.async_copy(ci_hbm.at[l2d.at[0]], gbuf0, sem0)
    cp1 = pltpu.async_copy(cs_hbm.at[l2d.at[0]], gbuf1, sem1)
    cp0.wait()
    pltpu.sync_copy(gbuf0, cir_o.at[pl.ds(base, CH)])
    cp1.wait()
    pltpu.sync_copy(gbuf1, csr_o.at[pl.ds(base, CH)])


def _sc_call(l, ci, cs):
    mesh = plsc.VectorSubcoreMesh(core_axis_name="c", subcore_axis_name="s",
                                  num_cores=NC)
    fn = pl.kernel(
        _sc_gather_phase,
        out_type=(
            jax.ShapeDtypeStruct((B, D), jnp.float32),
            jax.ShapeDtypeStruct((B, D), jnp.float32),
        ),
        mesh=mesh,
        scratch_types=[
            pltpu.VMEM((1, CH), jnp.int32),
            pltpu.VMEM((CH, D), jnp.float32),
            pltpu.VMEM((CH, D), jnp.float32),
            pltpu.SemaphoreType.DMA,
            pltpu.SemaphoreType.DMA,
        ],
    )
    return fn(l, ci, cs)


RB = 512
NBLK = B // RB


DA = D + 1


def _tc_a_phase(lcol_ref, lrow_ref, x_ref, aug_ref, xa_s):
    blk = pl.program_id(0)

    @pl.when(blk == 0)
    def _():
        xa_s[:, :D] = x_ref[...].astype(jnp.bfloat16)
        xa_s[:, D:DA] = jnp.ones((B, 1), jnp.bfloat16)

    lc = lcol_ref[...]
    lr = lrow_ref[...]
    mm = lc == lr
    aug_ref[...] = jax.lax.dot_general(
        mm.astype(jnp.bfloat16), xa_s[...],
        (((1,), (0,)), ((), ())), preferred_element_type=jnp.float32)


def _tc_a_call(l, x):
    return pl.pallas_call(
        _tc_a_phase,
        grid=(NBLK,),
        in_specs=[
            pl.BlockSpec((RB, 1), lambda i: (i, 0)),
            pl.BlockSpec((1, B), lambda i: (0, 0)),
            pl.BlockSpec((B, D), lambda i: (0, 0)),
        ],
        out_specs=pl.BlockSpec((RB, DA), lambda i: (i, 0)),
        out_shape=jax.ShapeDtypeStruct((B, DA), jnp.float32),
        scratch_shapes=[pltpu.VMEM((B, DA), jnp.bfloat16)],
    )(l.reshape(B, 1), l.reshape(1, B), x)


def _tc_b_phase(aug_ref, ci_ref, cs_ref, out_ref, sums):
    blk = pl.program_id(0)

    @pl.when(blk == 0)
    def _():
        sums[0] = 0.0
        sums[1] = 0.0

    aug = aug_ref[...]
    acc = aug[:, :D]
    cnt = aug[:, D:D + 1]
    inv = 1.0 / jnp.maximum(cnt, 1.0)
    mean = acc * inv
    upd = ci_ref[...] * MOM + mean * (1.0 - MOM)
    n2 = jnp.sum(upd * upd, axis=1, keepdims=True)
    uh = upd * jnp.where(n2 > 0, lax.rsqrt(n2), 1.0)
    diff = uh - cs_ref[...]
    d2 = jnp.sum(diff * diff, axis=1, keepdims=True)
    sums[0] += jnp.sum(d2 * inv)
    sums[1] += jnp.sum(inv)

    @pl.when(blk == NBLK - 1)
    def _():
        out_ref[...] = (sums[0] / jnp.maximum(sums[1], 1.0)).reshape(1, 1)


def _tc_b_call(aug, cir, csr):
    return pl.pallas_call(
        _tc_b_phase,
        grid=(NBLK,),
        in_specs=[
            pl.BlockSpec((RB, DA), lambda i: (i, 0)),
            pl.BlockSpec((RB, D), lambda i: (i, 0)),
            pl.BlockSpec((RB, D), lambda i: (i, 0)),
        ],
        out_specs=pl.BlockSpec((1, 1), lambda i: (0, 0)),
        out_shape=jax.ShapeDtypeStruct((1, 1), jnp.float32),
        scratch_shapes=[pltpu.SMEM((2,), jnp.float32)],
    )(aug, cir, csr)


def kernel(x, l, center_img, center_skt):
    cir, csr = _sc_call(l, center_img, center_skt)
    aug = _tc_a_call(l, x)
    loss = _tc_b_call(aug, cir, csr)
    return loss.reshape(())


# speedup vs baseline: 1.5654x; 1.0420x over previous
import jax
import jax.numpy as jnp
from jax import lax
from jax.experimental import pallas as pl
from jax.experimental.pallas import tpu as pltpu
from jax.experimental.pallas import tpu_sc as plsc

B = 4096
D = 384
MOM = 0.9

NC = 2
NS = 16
NW = NC * NS
N_EL = B // NW
CH = 128
LANES = 16


def _sc_gather_phase(l_hbm, ci_hbm, cs_hbm, cir_o, csr_o,
                     l2d, gbuf0, gbuf1, sem0, sem1):
    wid = lax.axis_index("s") * NC + lax.axis_index("c")
    base = wid * N_EL

    pltpu.sync_copy(l_hbm.at[pl.ds(base, CH)], l2d.at[0])

    cp0 = pltpu.async_copy(ci_hbm.at[l2d.at[0]], gbuf0, sem0)
    cp1 = pltpu.async_copy(cs_hbm.at[l2d.at[0]], gbuf1, sem1)
    cp0.wait()
    pltpu.sync_copy(gbuf0, cir_o.at[pl.ds(base, CH)])
    cp1.wait()
    pltpu.sync_copy(gbuf1, csr_o.at[pl.ds(base, CH)])


def _sc_call(l, ci, cs):
    mesh = plsc.VectorSubcoreMesh(core_axis_name="c", subcore_axis_name="s",
                                  num_cores=NC)
    fn = pl.kernel(
        _sc_gather_phase,
        out_type=(
            jax.ShapeDtypeStruct((B, D), jnp.float32),
            jax.ShapeDtypeStruct((B, D), jnp.float32),
        ),
        mesh=mesh,
        scratch_types=[
            pltpu.VMEM((1, CH), jnp.int32),
            pltpu.VMEM((CH, D), jnp.float32),
            pltpu.VMEM((CH, D), jnp.float32),
            pltpu.SemaphoreType.DMA,
            pltpu.SemaphoreType.DMA,
        ],
    )
    return fn(l, ci, cs)


RB = 512
NBLK = B // RB


DA = D + 1


def _tc_a_phase(lrows_ref, lrow_ref, x_ref, aug_ref, xa_s):
    blk = pl.program_id(0)

    @pl.when(blk == 0)
    def _():
        xa_s[:, :D] = x_ref[...].astype(jnp.bfloat16)
        xa_s[:, D:DA] = jnp.ones((B, 1), jnp.bfloat16)

    lc = jnp.transpose(lrows_ref[0], (1, 0))
    lr = lrow_ref[...]
    mm = lc == lr
    acc_aug = jax.lax.dot_general(
        mm.astype(jnp.bfloat16), xa_s[...],
        (((1,), (0,)), ((), ())), preferred_element_type=jnp.float32)
    acc = acc_aug[:, :D]
    cnt = acc_aug[:, D:D + 1]
    inv = 1.0 / jnp.maximum(cnt, 1.0)
    mean = acc * inv
    aug_ref[...] = jnp.concatenate([mean, inv], axis=1).astype(jnp.bfloat16)


def _tc_a_call(l, x):
    return pl.pallas_call(
        _tc_a_phase,
        grid=(NBLK,),
        in_specs=[
            pl.BlockSpec((1, 1, RB), lambda i: (i, 0, 0)),
            pl.BlockSpec((1, B), lambda i: (0, 0)),
            pl.BlockSpec((B, D), lambda i: (0, 0)),
        ],
        out_specs=pl.BlockSpec((RB, DA), lambda i: (i, 0)),
        out_shape=jax.ShapeDtypeStruct((B, DA), jnp.bfloat16),
        scratch_shapes=[pltpu.VMEM((B, DA), jnp.bfloat16)],
    )(l.reshape(NBLK, 1, RB), l.reshape(1, B), x)


def _tc_b_phase(aug_ref, ci_ref, cs_ref, out_ref, sums):
    blk = pl.program_id(0)

    @pl.when(blk == 0)
    def _():
        sums[0] = 0.0
        sums[1] = 0.0

    aug = aug_ref[...].astype(jnp.float32)
    mean = aug[:, :D]
    inv = aug[:, D:D + 1]
    upd = ci_ref[...] * MOM + mean * (1.0 - MOM)
    n2 = jnp.sum(upd * upd, axis=1, keepdims=True)
    uh = upd * jnp.where(n2 > 0, lax.rsqrt(n2), 1.0)
    diff = uh - cs_ref[...]
    d2 = jnp.sum(diff * diff, axis=1, keepdims=True)
    sums[0] += jnp.sum(d2 * inv)
    sums[1] += jnp.sum(inv)

    @pl.when(blk == NBLK - 1)
    def _():
        out_ref[...] = (sums[0] / jnp.maximum(sums[1], 1.0)).reshape(1, 1)


def _tc_b_call(aug, cir, csr):
    return pl.pallas_call(
        _tc_b_phase,
        grid=(NBLK,),
        in_specs=[
            pl.BlockSpec((RB, DA), lambda i: (i, 0)),
            pl.BlockSpec((RB, D), lambda i: (i, 0)),
            pl.BlockSpec((RB, D), lambda i: (i, 0)),
        ],
        out_specs=pl.BlockSpec((1, 1), lambda i: (0, 0)),
        out_shape=jax.ShapeDtypeStruct((1, 1), jnp.float32),
        scratch_shapes=[pltpu.SMEM((2,), jnp.float32)],
    )(aug, cir, csr)


def kernel(x, l, center_img, center_skt):
    cir, csr = _sc_call(l, center_img, center_skt)
    aug = _tc_a_call(l, x)
    loss = _tc_b_call(aug, cir, csr)
    return loss.reshape(())
